# Initial kernel scaffold; baseline (speedup 1.0000x reference)
import jax
import jax.numpy as jnp
from jax.experimental import pallas as pl


def kernel(input_fea, input_loc, pred_offset_w, residual_w, residual_b, residual_gamma, residual_beta):
    raise NotImplementedError("write your pallas kernel here")


# speedup vs baseline: 6.6840x; 6.6840x over previous
import functools

import jax
import jax.numpy as jnp
from jax import lax
from jax.experimental import pallas as pl

_NUM_NODE = 64
_NSAMPLE = 64
_RADIUS2 = 0.3 ** 2
_K_UP = 3


def _shift_lanes(x, s):
  r, n = x.shape
  return jnp.concatenate([jnp.zeros((r, s), x.dtype), x[:, : n - s]], axis=1)


def _shift_sublanes(x, s):
  r, n = x.shape
  return jnp.concatenate([jnp.zeros((s, n), x.dtype), x[: r - s, :]], axis=0)


def _cumsum_lanes_incl(x):
  _, n = x.shape
  s = 1
  while s < n:
    x = x + _shift_lanes(x, s)
    s *= 2
  return x


def _cumsum_sublanes_incl(x):
  r, _ = x.shape
  s = 1
  while s < r:
    x = x + _shift_sublanes(x, s)
    s *= 2
  return x


def _col(row):
  ones = jnp.ones((1, 1), row.dtype)
  return lax.dot_general(row, ones, (((0,), (0,)), ((), ())),
                         precision=lax.Precision.HIGHEST,
                         preferred_element_type=jnp.float32)


def _row(col):
  ones = jnp.ones((1, 1), col.dtype)
  return lax.dot_general(ones, col, (((1,), (1,)), ((), ())),
                         precision=lax.Precision.HIGHEST,
                         preferred_element_type=jnp.float32)


def _geom_body(loc_ref, fea_ref, wpo_ref, off_ref, mask_ref, wup_ref):
  n = loc_ref.shape[2]
  s_nodes = _NUM_NODE
  loc = loc_ref[0]
  fea = fea_ref[0]
  wpo = wpo_ref[...]

  g = lax.dot_general(wpo, fea, (((1,), (0,)), ((), ())),
                      precision=lax.Precision.HIGHEST,
                      preferred_element_type=jnp.float32)

  iota_n = lax.broadcasted_iota(jnp.int32, (1, n), 1)
  iota_s = lax.broadcasted_iota(jnp.int32, (1, s_nodes), 1)

  def fps_body(i, carry):
    dist, far, fpl, gf = carry
    oh = iota_n == far
    cen = jnp.sum(jnp.where(oh, loc, 0.0), axis=1, keepdims=True)
    gcol = jnp.sum(jnp.where(oh, g, 0.0), axis=1, keepdims=True)
    soh = iota_s == i
    fpl = fpl + jnp.where(soh, cen, 0.0)
    gf = gf + jnp.where(soh, gcol, 0.0)
    d = loc - cen
    dsq = (d[0:1] * d[0:1] + d[1:2] * d[1:2]) + d[2:3] * d[2:3]
    dist = jnp.minimum(dist, dsq)
    m = jnp.max(dist, axis=1, keepdims=True)
    far2 = jnp.min(jnp.where(dist == m, iota_n, n))
    return dist, far2, fpl, gf

  dist0 = jnp.full((1, n), 1e10, dtype=jnp.float32)
  fpl0 = jnp.zeros((3, s_nodes), jnp.float32)
  init = (dist0, jnp.int32(0), fpl0, fpl0)
  _, _, fpl, gf = lax.fori_loop(0, s_nodes, fps_body, init)

  fp_c = [_col(fpl[o:o + 1]) for o in range(3)]
  gf_c = [_col(gf[o:o + 1]) for o in range(3)]

  def _bf(x):
    return x.astype(jnp.bfloat16).astype(jnp.float32)

  bl = [_bf(loc[o:o + 1]) for o in range(3)]
  xnorm = (loc[0:1] * loc[0:1] + loc[1:2] * loc[1:2]) + loc[2:3] * loc[2:3]
  fpnorm = (fp_c[0] * fp_c[0] + fp_c[1] * fp_c[1]) + fp_c[2] * fp_c[2]
  cross = (_bf(fp_c[0]) * bl[0] + _bf(fp_c[1]) * bl[1]) + _bf(fp_c[2]) * bl[2]
  sq = fpnorm + xnorm - 2.0 * cross

  inr = sq <= _RADIUS2
  inr_f = inr.astype(jnp.float32)
  rank_ex = _cumsum_lanes_incl(inr_f) - inr_f
  cnt = jnp.sum(inr_f, axis=1, keepdims=True)
  sel = jnp.where(inr & (rank_ex < float(_NSAMPLE)), 1.0, 0.0)
  first = jnp.where(inr & (rank_ex == 0.0), 1.0, 0.0)
  pad = jnp.maximum(float(_NSAMPLE) - cnt, 0.0)

  off_rows = []
  nl_c = []
  for o in range(3):
    gd = jnp.tanh(g[o:o + 1] - gf_c[o])
    ld = loc[o:o + 1] - fp_c[o]
    f = gd * ld
    s_sel = jnp.sum(f * sel, axis=1, keepdims=True)
    f_n0 = jnp.sum(f * first, axis=1, keepdims=True)
    off = (s_sel + pad * f_n0) * (1.0 / _NSAMPLE)
    off_rows.append(_row(off))
    nl_c.append(fp_c[o] + off)
  off_ref[0] = jnp.concatenate(off_rows, axis=0)

  nlnorm = (nl_c[0] * nl_c[0] + nl_c[1] * nl_c[1]) + nl_c[2] * nl_c[2]
  cross2 = (_bf(nl_c[0]) * bl[0] + _bf(nl_c[1]) * bl[1]) + _bf(nl_c[2]) * bl[2]
  sq2 = nlnorm + xnorm - 2.0 * cross2

  keys = lax.bitcast_convert_type(jnp.maximum(sq2, 0.0), jnp.int32)
  cur = jnp.zeros((s_nodes, 1), jnp.int32)
  for bit in range(30, -1, -1):
    trial = cur | jnp.int32(1 << bit)
    cnt_lt = jnp.sum(jnp.where(keys < trial, 1.0, 0.0), axis=1, keepdims=True)
    cur = jnp.where(cnt_lt < float(_NSAMPLE), trial, cur)
  thr = cur
  lt = jnp.where(keys < thr, 1.0, 0.0)
  c1 = jnp.sum(lt, axis=1, keepdims=True)
  tie = jnp.where(keys == thr, 1.0, 0.0)
  tierank = _cumsum_lanes_incl(tie) - tie
  need = float(_NSAMPLE) - c1
  mask_ref[0] = lt + tie * jnp.where(tierank < need, 1.0, 0.0)

  work = sq2
  wmat = jnp.zeros((s_nodes, n), jnp.float32)
  wsum = jnp.zeros((1, n), jnp.float32)
  for _ in range(_K_UP):
    m = jnp.min(work, axis=0, keepdims=True)
    eq = jnp.where(work == m, 1.0, 0.0)
    req = _cumsum_sublanes_incl(eq) - eq
    pick = (eq > 0.0) & (req == 0.0)
    wj = 1.0 / jnp.maximum(m, 1e-10)
    wmat = wmat + jnp.where(pick, wj, 0.0)
    wsum = wsum + wj
    work = jnp.where(pick, jnp.float32(jnp.inf), work)
  wup_ref[0] = wmat / wsum


def _res_body(fea_ref, w_ref, b_ref, y_ref, st_ref):
  b = pl.program_id(0)

  @pl.when(b == 0)
  def _():
    st_ref[...] = jnp.zeros_like(st_ref)

  fea = fea_ref[0]
  y = lax.dot_general(w_ref[...], fea, (((1,), (0,)), ((), ())),
                      precision=lax.Precision.DEFAULT,
                      preferred_element_type=jnp.float32)
  y = y + b_ref[...]
  y_ref[0] = y
  st_ref[:, 0:1] = st_ref[:, 0:1] + jnp.sum(y, axis=1, keepdims=True)
  st_ref[:, 1:2] = st_ref[:, 1:2] + jnp.sum(y * y, axis=1, keepdims=True)


def _fuse_body(y_ref, fea_ref, mask_ref, wup_ref, a_ref, c_ref,
               out_ref, nf_ref):
  c_ch = y_ref.shape[1]
  s_nodes = _NUM_NODE
  rf = jnp.maximum(y_ref[0] * a_ref[...] + c_ref[...], 0.0)
  iota_s = lax.broadcasted_iota(jnp.int32, (1, s_nodes), 1)
  mask_v = mask_ref[0]

  def sbody(s, nf):
    soh = (iota_s == s).astype(jnp.float32)
    row = lax.dot_general(soh, mask_v, (((1,), (0,)), ((), ())),
                          precision=lax.Precision.HIGHEST,
                          preferred_element_type=jnp.float32)
    v = jnp.max(rf * row, axis=1, keepdims=True)
    return nf + jnp.where(iota_s == s, v, 0.0)

  nf = lax.fori_loop(0, s_nodes, sbody, jnp.zeros((c_ch, s_nodes),
                                                  jnp.float32))
  nf_ref[0] = nf
  interp = lax.dot_general(nf, wup_ref[0], (((1,), (0,)), ((), ())),
                           precision=lax.Precision.HIGHEST,
                           preferred_element_type=jnp.float32)
  out_ref[0, :c_ch, :] = fea_ref[0]
  out_ref[0, c_ch:, :] = interp


def kernel(input_fea, input_loc, pred_offset_w, residual_w, residual_b,
           residual_gamma, residual_beta):
  b_n, c_ch, n, _ = input_fea.shape
  s_nodes = _NUM_NODE
  fea = input_fea[..., 0]

  off, mask, wup = pl.pallas_call(
      _geom_body,
      grid=(b_n,),
      in_specs=[
          pl.BlockSpec((1, 3, n), lambda b: (b, 0, 0)),
          pl.BlockSpec((1, c_ch, n), lambda b: (b, 0, 0)),
          pl.BlockSpec((3, c_ch), lambda b: (0, 0)),
      ],
      out_specs=[
          pl.BlockSpec((1, 3, s_nodes), lambda b: (b, 0, 0)),
          pl.BlockSpec((1, s_nodes, n), lambda b: (b, 0, 0)),
          pl.BlockSpec((1, s_nodes, n), lambda b: (b, 0, 0)),
      ],
      out_shape=[
          jax.ShapeDtypeStruct((b_n, 3, s_nodes), jnp.float32),
          jax.ShapeDtypeStruct((b_n, s_nodes, n), jnp.float32),
          jax.ShapeDtypeStruct((b_n, s_nodes, n), jnp.float32),
      ],
  )(input_loc, fea, pred_offset_w)

  y, stats = pl.pallas_call(
      _res_body,
      grid=(b_n,),
      in_specs=[
          pl.BlockSpec((1, c_ch, n), lambda b: (b, 0, 0)),
          pl.BlockSpec((c_ch, c_ch), lambda b: (0, 0)),
          pl.BlockSpec((c_ch, 1), lambda b: (0, 0)),
      ],
      out_specs=[
          pl.BlockSpec((1, c_ch, n), lambda b: (b, 0, 0)),
          pl.BlockSpec((c_ch, 128), lambda b: (0, 0)),
      ],
      out_shape=[
          jax.ShapeDtypeStruct((b_n, c_ch, n), jnp.float32),
          jax.ShapeDtypeStruct((c_ch, 128), jnp.float32),
      ],
  )(fea, residual_w, residual_b.reshape(c_ch, 1))

  count = float(b_n * n)
  mean = stats[:, 0] / count
  var = stats[:, 1] / count - mean * mean
  a = residual_gamma / jnp.sqrt(var + 1e-5)
  c = residual_beta - mean * a

  out_fea, node_fea = pl.pallas_call(
      _fuse_body,
      grid=(b_n,),
      in_specs=[
          pl.BlockSpec((1, c_ch, n), lambda b: (b, 0, 0)),
          pl.BlockSpec((1, c_ch, n), lambda b: (b, 0, 0)),
          pl.BlockSpec((1, s_nodes, n), lambda b: (b, 0, 0)),
          pl.BlockSpec((1, s_nodes, n), lambda b: (b, 0, 0)),
          pl.BlockSpec((c_ch, 1), lambda b: (0, 0)),
          pl.BlockSpec((c_ch, 1), lambda b: (0, 0)),
      ],
      out_specs=[
          pl.BlockSpec((1, 2 * c_ch, n), lambda b: (b, 0, 0)),
          pl.BlockSpec((1, c_ch, s_nodes), lambda b: (b, 0, 0)),
      ],
      out_shape=[
          jax.ShapeDtypeStruct((b_n, 2 * c_ch, n), jnp.float32),
          jax.ShapeDtypeStruct((b_n, c_ch, s_nodes), jnp.float32),
      ],
  )(y, fea, mask, wup, a.reshape(c_ch, 1), c.reshape(c_ch, 1))

  return (out_fea[..., None], node_fea[..., None], off)


# speedup vs baseline: 6.6850x; 1.0002x over previous
import jax
import jax.numpy as jnp
from jax import lax
from jax.experimental import pallas as pl

_NUM_NODE = 64
_NSAMPLE = 64
_RADIUS2 = 0.3 ** 2
_K_UP = 3


def _shift_lanes(x, s):
  r, n = x.shape
  return jnp.concatenate([jnp.zeros((r, s), x.dtype), x[:, : n - s]], axis=1)


def _shift_sublanes(x, s):
  r, n = x.shape
  return jnp.concatenate([jnp.zeros((s, n), x.dtype), x[: r - s, :]], axis=0)


def _cumsum_lanes_incl(x):
  _, n = x.shape
  s = 1
  while s < n:
    x = x + _shift_lanes(x, s)
    s *= 2
  return x


def _cumsum_sublanes_incl(x):
  r, _ = x.shape
  s = 1
  while s < r:
    x = x + _shift_sublanes(x, s)
    s *= 2
  return x


def _col(row):
  ones = jnp.ones((1, 1), row.dtype)
  return lax.dot_general(row, ones, (((0,), (0,)), ((), ())),
                         precision=lax.Precision.HIGHEST,
                         preferred_element_type=jnp.float32)


def _row(col):
  ones = jnp.ones((1, 1), col.dtype)
  return lax.dot_general(ones, col, (((1,), (1,)), ((), ())),
                         precision=lax.Precision.HIGHEST,
                         preferred_element_type=jnp.float32)


def _geom_body(loc_ref, fea_ref, wpo_ref, off_ref, mask_ref, wup_ref):
  n = loc_ref.shape[2]
  s_nodes = _NUM_NODE
  loc = loc_ref[0]
  fea = fea_ref[0]
  wpo = wpo_ref[...]

  g = lax.dot_general(wpo, fea, (((1,), (0,)), ((), ())),
                      precision=lax.Precision.HIGHEST,
                      preferred_element_type=jnp.float32)

  iota_n = lax.broadcasted_iota(jnp.int32, (1, n), 1)
  iota_s = lax.broadcasted_iota(jnp.int32, (1, s_nodes), 1)

  def fps_body(i, carry):
    dist, far, fpl, gf = carry
    oh = iota_n == far
    cen = jnp.sum(jnp.where(oh, loc, 0.0), axis=1, keepdims=True)
    gcol = jnp.sum(jnp.where(oh, g, 0.0), axis=1, keepdims=True)
    soh = iota_s == i
    fpl = fpl + jnp.where(soh, cen, 0.0)
    gf = gf + jnp.where(soh, gcol, 0.0)
    d = loc - cen
    dsq = (d[0:1] * d[0:1] + d[1:2] * d[1:2]) + d[2:3] * d[2:3]
    dist = jnp.minimum(dist, dsq)
    m = jnp.max(dist, axis=1, keepdims=True)
    far2 = jnp.min(jnp.where(dist == m, iota_n, n))
    return dist, far2, fpl, gf

  dist0 = jnp.full((1, n), 1e10, dtype=jnp.float32)
  fpl0 = jnp.zeros((3, s_nodes), jnp.float32)
  init = (dist0, jnp.int32(0), fpl0, fpl0)
  _, _, fpl, gf = lax.fori_loop(0, s_nodes, fps_body, init)

  fp_c = [_col(fpl[o:o + 1]) for o in range(3)]
  gf_c = [_col(gf[o:o + 1]) for o in range(3)]

  def _bf(x):
    return x.astype(jnp.bfloat16).astype(jnp.float32)

  bl = [_bf(loc[o:o + 1]) for o in range(3)]
  xnorm = (loc[0:1] * loc[0:1] + loc[1:2] * loc[1:2]) + loc[2:3] * loc[2:3]
  fpnorm = (fp_c[0] * fp_c[0] + fp_c[1] * fp_c[1]) + fp_c[2] * fp_c[2]
  cross = (_bf(fp_c[0]) * bl[0] + _bf(fp_c[1]) * bl[1]) + _bf(fp_c[2]) * bl[2]
  sq = fpnorm + xnorm - 2.0 * cross

  inr = sq <= _RADIUS2
  inr_f = inr.astype(jnp.float32)
  rank_ex = _cumsum_lanes_incl(inr_f) - inr_f
  cnt = jnp.sum(inr_f, axis=1, keepdims=True)
  sel = jnp.where(inr & (rank_ex < float(_NSAMPLE)), 1.0, 0.0)
  first = jnp.where(inr & (rank_ex == 0.0), 1.0, 0.0)
  pad = jnp.maximum(float(_NSAMPLE) - cnt, 0.0)

  off_rows = []
  nl_c = []
  for o in range(3):
    gd = jnp.tanh(g[o:o + 1] - gf_c[o])
    ld = loc[o:o + 1] - fp_c[o]
    f = gd * ld
    s_sel = jnp.sum(f * sel, axis=1, keepdims=True)
    f_n0 = jnp.sum(f * first, axis=1, keepdims=True)
    off = (s_sel + pad * f_n0) * (1.0 / _NSAMPLE)
    off_rows.append(_row(off))
    nl_c.append(fp_c[o] + off)
  off_ref[0] = jnp.concatenate(off_rows, axis=0)

  nlnorm = (nl_c[0] * nl_c[0] + nl_c[1] * nl_c[1]) + nl_c[2] * nl_c[2]
  cross2 = (_bf(nl_c[0]) * bl[0] + _bf(nl_c[1]) * bl[1]) + _bf(nl_c[2]) * bl[2]
  sq2 = nlnorm + xnorm - 2.0 * cross2

  keys = lax.bitcast_convert_type(jnp.maximum(sq2, 0.0), jnp.int32)
  cur = jnp.zeros((s_nodes, 1), jnp.int32)
  for bit in range(30, -1, -1):
    trial = cur | jnp.int32(1 << bit)
    cnt_lt = jnp.sum(jnp.where(keys < trial, 1.0, 0.0), axis=1, keepdims=True)
    cur = jnp.where(cnt_lt < float(_NSAMPLE), trial, cur)
  thr = cur
  lt = jnp.where(keys < thr, 1.0, 0.0)
  c1 = jnp.sum(lt, axis=1, keepdims=True)
  tie = jnp.where(keys == thr, 1.0, 0.0)
  tierank = _cumsum_lanes_incl(tie) - tie
  need = float(_NSAMPLE) - c1
  mask_ref[0] = lt + tie * jnp.where(tierank < need, 1.0, 0.0)

  work = sq2
  wmat = jnp.zeros((s_nodes, n), jnp.float32)
  wsum = jnp.zeros((1, n), jnp.float32)
  for _ in range(_K_UP):
    m = jnp.min(work, axis=0, keepdims=True)
    eq = jnp.where(work == m, 1.0, 0.0)
    req = _cumsum_sublanes_incl(eq) - eq
    pick = (eq > 0.0) & (req == 0.0)
    wj = 1.0 / jnp.maximum(m, 1e-10)
    wmat = wmat + jnp.where(pick, wj, 0.0)
    wsum = wsum + wj
    work = jnp.where(pick, jnp.float32(jnp.inf), work)
  wup_ref[0] = wmat / wsum


def _res_body(fea_ref, w_ref, b_ref, y_ref, st_ref):
  b = pl.program_id(0)

  @pl.when(b == 0)
  def _():
    st_ref[...] = jnp.zeros_like(st_ref)

  fea = fea_ref[0]
  y = lax.dot_general(w_ref[...], fea, (((1,), (0,)), ((), ())),
                      precision=lax.Precision.DEFAULT,
                      preferred_element_type=jnp.float32)
  y = y + b_ref[...]
  y_ref[0] = y
  st_ref[:, 0:1] = st_ref[:, 0:1] + jnp.sum(y, axis=1, keepdims=True)
  st_ref[:, 1:2] = st_ref[:, 1:2] + jnp.sum(y * y, axis=1, keepdims=True)


def _fuse_body(y_ref, fea_ref, mask_ref, wup_ref, a_ref, c_ref,
               out_ref, nf_ref):
  c_ch = y_ref.shape[1]
  s_nodes = _NUM_NODE
  rf = jnp.maximum(y_ref[0] * a_ref[...] + c_ref[...], 0.0)
  iota_s = lax.broadcasted_iota(jnp.int32, (1, s_nodes), 1)
  mask_v = mask_ref[0]

  def sbody(s, nf):
    soh = (iota_s == s).astype(jnp.float32)
    row = lax.dot_general(soh, mask_v, (((1,), (0,)), ((), ())),
                          precision=lax.Precision.HIGHEST,
                          preferred_element_type=jnp.float32)
    v = jnp.max(rf * row, axis=1, keepdims=True)
    return nf + jnp.where(iota_s == s, v, 0.0)

  nf = lax.fori_loop(0, s_nodes, sbody, jnp.zeros((c_ch, s_nodes),
                                                  jnp.float32))
  nf_ref[0] = nf
  interp = lax.dot_general(nf, wup_ref[0], (((1,), (0,)), ((), ())),
                           precision=lax.Precision.HIGHEST,
                           preferred_element_type=jnp.float32)
  out_ref[0, :c_ch, :] = fea_ref[0]
  out_ref[0, c_ch:, :] = interp


def kernel(input_fea, input_loc, pred_offset_w, residual_w, residual_b,
           residual_gamma, residual_beta):
  b_n, c_ch, n, _ = input_fea.shape
  s_nodes = _NUM_NODE
  fea = input_fea[..., 0]

  off, mask, wup = pl.pallas_call(
      _geom_body,
      grid=(b_n,),
      in_specs=[
          pl.BlockSpec((1, 3, n), lambda b: (b, 0, 0)),
          pl.BlockSpec((1, c_ch, n), lambda b: (b, 0, 0)),
          pl.BlockSpec((3, c_ch), lambda b: (0, 0)),
      ],
      out_specs=[
          pl.BlockSpec((1, 3, s_nodes), lambda b: (b, 0, 0)),
          pl.BlockSpec((1, s_nodes, n), lambda b: (b, 0, 0)),
          pl.BlockSpec((1, s_nodes, n), lambda b: (b, 0, 0)),
      ],
      out_shape=[
          jax.ShapeDtypeStruct((b_n, 3, s_nodes), jnp.float32),
          jax.ShapeDtypeStruct((b_n, s_nodes, n), jnp.float32),
          jax.ShapeDtypeStruct((b_n, s_nodes, n), jnp.float32),
      ],
  )(input_loc, fea, pred_offset_w)

  y, stats = pl.pallas_call(
      _res_body,
      grid=(b_n,),
      in_specs=[
          pl.BlockSpec((1, c_ch, n), lambda b: (b, 0, 0)),
          pl.BlockSpec((c_ch, c_ch), lambda b: (0, 0)),
          pl.BlockSpec((c_ch, 1), lambda b: (0, 0)),
      ],
      out_specs=[
          pl.BlockSpec((1, c_ch, n), lambda b: (b, 0, 0)),
          pl.BlockSpec((c_ch, 128), lambda b: (0, 0)),
      ],
      out_shape=[
          jax.ShapeDtypeStruct((b_n, c_ch, n), jnp.float32),
          jax.ShapeDtypeStruct((c_ch, 128), jnp.float32),
      ],
  )(fea, residual_w, residual_b.reshape(c_ch, 1))

  count = float(b_n * n)
  mean = stats[:, 0] / count
  var = stats[:, 1] / count - mean * mean
  a = residual_gamma / jnp.sqrt(var + 1e-5)
  c = residual_beta - mean * a

  out_fea, node_fea = pl.pallas_call(
      _fuse_body,
      grid=(b_n,),
      in_specs=[
          pl.BlockSpec((1, c_ch, n), lambda b: (b, 0, 0)),
          pl.BlockSpec((1, c_ch, n), lambda b: (b, 0, 0)),
          pl.BlockSpec((1, s_nodes, n), lambda b: (b, 0, 0)),
          pl.BlockSpec((1, s_nodes, n), lambda b: (b, 0, 0)),
          pl.BlockSpec((c_ch, 1), lambda b: (0, 0)),
          pl.BlockSpec((c_ch, 1), lambda b: (0, 0)),
      ],
      out_specs=[
          pl.BlockSpec((1, 2 * c_ch, n), lambda b: (b, 0, 0)),
          pl.BlockSpec((1, c_ch, s_nodes), lambda b: (b, 0, 0)),
      ],
      out_shape=[
          jax.ShapeDtypeStruct((b_n, 2 * c_ch, n), jnp.float32),
          jax.ShapeDtypeStruct((b_n, c_ch, s_nodes), jnp.float32),
      ],
  )(y, fea, mask, wup, a.reshape(c_ch, 1), c.reshape(c_ch, 1))

  return (out_fea[..., None], node_fea[..., None], off)
